# Initial kernel scaffold; baseline (speedup 1.0000x reference)
import jax
import jax.numpy as jnp
from jax.experimental import pallas as pl


def kernel(x, B, Pi):
    raise NotImplementedError("write your pallas kernel here")


# speedup vs baseline: 1.9757x; 1.9757x over previous
import functools

import jax
import jax.numpy as jnp
from jax import lax
from jax.experimental import pallas as pl
from jax.experimental.pallas import tpu as pltpu
from jax.experimental.pallas import tpu_sc as plsc

_CHUNK = 80


def _table_body(bt_ref, pi_ref, out_ref):
    bt = bt_ref[...]
    e = jnp.exp(bt - jnp.max(bt, axis=0, keepdims=True))
    s = jnp.sum(e, axis=0, keepdims=True)
    pi = pi_ref[...]
    pe = jnp.exp(pi - jnp.max(pi, axis=1, keepdims=True))
    ps = jnp.sum(pe, axis=1, keepdims=True)
    out_ref[...] = e * (pe / (s * ps))


def kernel(x, B, Pi):
    c_dim, m_dim = B.shape
    n_dim = x.shape[0]
    ch = _CHUNK
    n_chunks = n_dim // ch

    wt = pl.pallas_call(
        _table_body,
        out_shape=jax.ShapeDtypeStruct((m_dim, c_dim), jnp.float32),
    )(B.T, Pi.reshape(1, c_dim))

    mesh = plsc.VectorSubcoreMesh(core_axis_name="c", subcore_axis_name="s")
    nw = mesh.num_cores * mesh.num_subcores
    n_iters = -(-n_chunks // nw)

    @functools.partial(
        pl.kernel,
        out_type=jax.ShapeDtypeStruct((n_dim, c_dim), jnp.float32),
        mesh=mesh,
        scratch_types=[
            pltpu.VMEM((ch,), jnp.int32),
            pltpu.VMEM((ch, c_dim), jnp.float32),
            pltpu.SemaphoreType.DMA,
        ],
    )
    def _gather(wt_hbm, x_hbm, out_hbm, idx_v, rows_v, sem):
        cid = lax.axis_index("c")
        sid = lax.axis_index("s")
        wid = sid * mesh.num_cores + cid

        def body(j, carry):
            g = wid + j * nw

            @pl.when(g < n_chunks)
            def _():
                off = g * ch
                pltpu.sync_copy(x_hbm.at[pl.ds(off, ch)], idx_v)
                pltpu.async_copy(wt_hbm.at[idx_v], rows_v, sem).wait()
                pltpu.sync_copy(rows_v, out_hbm.at[pl.ds(off, ch)])

            return carry

        lax.fori_loop(0, n_iters, body, 0)

    return _gather(wt, x)


# speedup vs baseline: 2.2571x; 1.1424x over previous
import functools

import jax
import jax.numpy as jnp
from jax import lax
from jax.experimental import pallas as pl
from jax.experimental.pallas import tpu as pltpu
from jax.experimental.pallas import tpu_sc as plsc

_CHUNK = 80


def _table_body(bt_ref, pi_ref, out_ref):
    bt = bt_ref[...]
    e = jnp.exp(bt - jnp.max(bt, axis=0, keepdims=True))
    s = jnp.sum(e, axis=0, keepdims=True)
    pi = pi_ref[...]
    pe = jnp.exp(pi - jnp.max(pi, axis=1, keepdims=True))
    ps = jnp.sum(pe, axis=1, keepdims=True)
    out_ref[...] = e * (pe / (s * ps))


def kernel(x, B, Pi):
    c_dim, m_dim = B.shape
    n_dim = x.shape[0]
    ch = _CHUNK
    n_chunks = n_dim // ch

    wt = pl.pallas_call(
        _table_body,
        out_shape=jax.ShapeDtypeStruct((m_dim, c_dim), jnp.float32),
    )(B.T, Pi.reshape(1, c_dim))

    mesh = plsc.VectorSubcoreMesh(core_axis_name="c", subcore_axis_name="s")
    nw = mesh.num_cores * mesh.num_subcores
    n_iters = -(-n_chunks // nw)
    assert n_iters % 2 == 0 and n_chunks >= 2 * nw

    @functools.partial(
        pl.kernel,
        out_type=jax.ShapeDtypeStruct((n_dim, c_dim), jnp.float32),
        mesh=mesh,
        scratch_types=[
            pltpu.VMEM((ch,), jnp.int32),
            pltpu.VMEM((ch,), jnp.int32),
            pltpu.VMEM((ch, c_dim), jnp.float32),
            pltpu.VMEM((ch, c_dim), jnp.float32),
            pltpu.SemaphoreType.DMA,
            pltpu.SemaphoreType.DMA,
            pltpu.SemaphoreType.DMA,
            pltpu.SemaphoreType.DMA,
            pltpu.SemaphoreType.DMA,
            pltpu.SemaphoreType.DMA,
        ],
    )
    def _gather(wt_hbm, x_hbm, out_hbm, idx0, idx1, rows0, rows1,
                isem0, isem1, gsem0, gsem1, osem0, osem1):
        idx = (idx0, idx1)
        rows = (rows0, rows1)
        isem = (isem0, isem1)
        gsem = (gsem0, gsem1)
        osem = (osem0, osem1)
        cid = lax.axis_index("c")
        sid = lax.axis_index("s")
        wid = sid * mesh.num_cores + cid

        for b in range(2):
            pltpu.async_copy(
                x_hbm.at[pl.ds((wid + b * nw) * ch, ch)], idx[b], isem[b])

        def body(j2, carry):
            for b in range(2):
                j = 2 * j2 + b
                g = wid + j * nw

                @pl.when(g < n_chunks)
                def _(b=b, j=j, g=g):
                    off = g * ch
                    @pl.when(j >= 2)
                    def _():
                        pltpu.make_async_copy(
                            rows[b], out_hbm.at[pl.ds(off, ch)], osem[b]
                        ).wait()

                    pltpu.make_async_copy(
                        x_hbm.at[pl.ds(off, ch)], idx[b], isem[b]).wait()
                    pltpu.async_copy(wt_hbm.at[idx[b]], rows[b], gsem[b]).wait()
                    pltpu.async_copy(
                        rows[b], out_hbm.at[pl.ds(off, ch)], osem[b])

                    g2 = g + 2 * nw

                    @pl.when(g2 < n_chunks)
                    def _():
                        pltpu.async_copy(
                            x_hbm.at[pl.ds(g2 * ch, ch)], idx[b], isem[b])

            return carry

        lax.fori_loop(0, n_iters // 2, body, 0)

        for b in range(2):
            pltpu.make_async_copy(
                rows[b], out_hbm.at[pl.ds(0, ch)], osem[b]).wait()

    return _gather(wt, x)


# speedup vs baseline: 2.3687x; 1.0494x over previous
import functools

import jax
import jax.numpy as jnp
from jax import lax
from jax.experimental import pallas as pl
from jax.experimental.pallas import tpu as pltpu
from jax.experimental.pallas import tpu_sc as plsc

_CHUNK = 80


def _table_body(bt_ref, pi_ref, out_ref):
    bt = bt_ref[...]
    e = jnp.exp(bt - jnp.max(bt, axis=0, keepdims=True))
    s = jnp.sum(e, axis=0, keepdims=True)
    pi = pi_ref[...]
    pe = jnp.exp(pi - jnp.max(pi, axis=1, keepdims=True))
    ps = jnp.sum(pe, axis=1, keepdims=True)
    out_ref[...] = e * (pe / (s * ps))


def kernel(x, B, Pi):
    c_dim, m_dim = B.shape
    n_dim = x.shape[0]
    ch = _CHUNK
    n_chunks = n_dim // ch

    wt = pl.pallas_call(
        _table_body,
        out_shape=jax.ShapeDtypeStruct((m_dim, c_dim), jnp.float32),
    )(B.T, Pi.reshape(1, c_dim))

    mesh = plsc.VectorSubcoreMesh(core_axis_name="c", subcore_axis_name="s")
    nw = mesh.num_cores * mesh.num_subcores
    n_iters = -(-n_chunks // nw)
    nbuf = 4
    assert n_iters % nbuf == 0 and n_chunks >= nbuf * nw

    @functools.partial(
        pl.kernel,
        out_type=jax.ShapeDtypeStruct((n_dim, c_dim), jnp.float32),
        mesh=mesh,
        scratch_types=[
            [pltpu.VMEM((ch,), jnp.int32)] * nbuf,
            [pltpu.VMEM((ch, c_dim), jnp.float32)] * nbuf,
            [pltpu.SemaphoreType.DMA] * nbuf,
            [pltpu.SemaphoreType.DMA] * nbuf,
            [pltpu.SemaphoreType.DMA] * nbuf,
        ],
    )
    def _gather(wt_hbm, x_hbm, out_hbm, idx, rows, isem, gsem, osem):
        cid = lax.axis_index("c")
        sid = lax.axis_index("s")
        wid = sid * mesh.num_cores + cid

        def active(c):
            return (wid + c * nw) < n_chunks

        def off_of(c):
            return (wid + c * nw) * ch

        for b in range(nbuf):
            pltpu.async_copy(x_hbm.at[pl.ds(off_of(b), ch)], idx[b], isem[b])
        for b in range(nbuf - 1):
            pltpu.make_async_copy(
                x_hbm.at[pl.ds(off_of(b), ch)], idx[b], isem[b]).wait()
            pltpu.async_copy(wt_hbm.at[idx[b]], rows[b], gsem[b])

        def body(jq, carry):
            for b in range(nbuf):
                j = nbuf * jq + b
                g = wid + j * nw
                off = g * ch

                @pl.when(g < n_chunks)
                def _(b=b, off=off):
                    pltpu.make_async_copy(
                        wt_hbm.at[idx[b]], rows[b], gsem[b]).wait()
                    pltpu.async_copy(
                        rows[b], out_hbm.at[pl.ds(off, ch)], osem[b])

                    @pl.when((off + nbuf * nw * ch) < n_chunks * ch)
                    def _():
                        pltpu.async_copy(
                            x_hbm.at[pl.ds(off + nbuf * nw * ch, ch)],
                            idx[b], isem[b])

                b3 = (b + nbuf - 1) % nbuf
                c3 = j + nbuf - 1

                @pl.when(active(c3))
                def _(b3=b3, c3=c3, b=b, jq=jq):
                    pltpu.make_async_copy(
                        x_hbm.at[pl.ds(off_of(c3), ch)], idx[b3], isem[b3]
                    ).wait()
                    if b == 0:
                        @pl.when(jq >= 1)
                        def _():
                            pltpu.make_async_copy(
                                rows[b3], out_hbm.at[pl.ds(0, ch)], osem[b3]
                            ).wait()
                    else:
                        pltpu.make_async_copy(
                            rows[b3], out_hbm.at[pl.ds(0, ch)], osem[b3]
                        ).wait()
                    pltpu.async_copy(wt_hbm.at[idx[b3]], rows[b3], gsem[b3])

            return carry

        lax.fori_loop(0, n_iters // nbuf, body, 0)

        for b in range(nbuf):
            pltpu.make_async_copy(
                rows[b], out_hbm.at[pl.ds(0, ch)], osem[b]).wait()

    return _gather(wt, x)


# speedup vs baseline: 2.9821x; 1.2590x over previous
import functools

import jax
import jax.numpy as jnp
from jax import lax
from jax.experimental import pallas as pl
from jax.experimental.pallas import tpu as pltpu
from jax.experimental.pallas import tpu_sc as plsc

_CHUNK = 80


_REP = 8


def _table_body(bt_ref, pi_ref, out_ref):
    bt = bt_ref[...]
    e = jnp.exp(bt - jnp.max(bt, axis=0, keepdims=True))
    s = jnp.sum(e, axis=0, keepdims=True)
    pi = pi_ref[...]
    pe = jnp.exp(pi - jnp.max(pi, axis=1, keepdims=True))
    ps = jnp.sum(pe, axis=1, keepdims=True)
    w = e * (pe / (s * ps))
    out_ref[...] = jnp.concatenate([w] * _REP, axis=1)


def kernel(x, B, Pi):
    c_dim, m_dim = B.shape
    n_dim = x.shape[0]
    ch = _CHUNK
    n_chunks = n_dim // ch

    wt = pl.pallas_call(
        _table_body,
        out_shape=jax.ShapeDtypeStruct((m_dim, _REP * c_dim), jnp.float32),
    )(B.T, Pi.reshape(1, c_dim))
    wt = wt.reshape(m_dim * _REP, c_dim)

    mesh = plsc.VectorSubcoreMesh(core_axis_name="c", subcore_axis_name="s")
    nw = mesh.num_cores * mesh.num_subcores
    n_iters = -(-n_chunks // nw)
    nbuf = 4
    assert n_iters % nbuf == 0 and n_chunks >= nbuf * nw

    @functools.partial(
        pl.kernel,
        out_type=jax.ShapeDtypeStruct((n_dim, c_dim), jnp.float32),
        mesh=mesh,
        scratch_types=[
            [pltpu.VMEM((ch,), jnp.int32)] * nbuf,
            [pltpu.VMEM((ch, c_dim), jnp.float32)] * nbuf,
            [pltpu.SemaphoreType.DMA] * nbuf,
            [pltpu.SemaphoreType.DMA] * nbuf,
            [pltpu.SemaphoreType.DMA] * nbuf,
        ],
    )
    def _gather(wt_hbm, x_hbm, out_hbm, idx, rows, isem, gsem, osem):
        cid = lax.axis_index("c")
        sid = lax.axis_index("s")
        wid = sid * mesh.num_cores + cid

        def adjust(b):
            rep = lax.iota(jnp.int32, 16) & (_REP - 1)
            for i in range(ch // 16):
                v = idx[b][pl.ds(16 * i, 16)]
                idx[b][pl.ds(16 * i, 16)] = v * _REP + rep

        def active(c):
            return (wid + c * nw) < n_chunks

        def off_of(c):
            return (wid + c * nw) * ch

        for b in range(nbuf):
            pltpu.async_copy(x_hbm.at[pl.ds(off_of(b), ch)], idx[b], isem[b])
        for b in range(nbuf - 1):
            pltpu.make_async_copy(
                x_hbm.at[pl.ds(off_of(b), ch)], idx[b], isem[b]).wait()
            adjust(b)
            pltpu.async_copy(wt_hbm.at[idx[b]], rows[b], gsem[b])

        def body(jq, carry):
            for b in range(nbuf):
                j = nbuf * jq + b
                g = wid + j * nw
                off = g * ch

                @pl.when(g < n_chunks)
                def _(b=b, off=off):
                    pltpu.make_async_copy(
                        wt_hbm.at[idx[b]], rows[b], gsem[b]).wait()
                    pltpu.async_copy(
                        rows[b], out_hbm.at[pl.ds(off, ch)], osem[b])

                    @pl.when((off + nbuf * nw * ch) < n_chunks * ch)
                    def _():
                        pltpu.async_copy(
                            x_hbm.at[pl.ds(off + nbuf * nw * ch, ch)],
                            idx[b], isem[b])

                b3 = (b + nbuf - 1) % nbuf
                c3 = j + nbuf - 1

                @pl.when(active(c3))
                def _(b3=b3, c3=c3, b=b, jq=jq):
                    pltpu.make_async_copy(
                        x_hbm.at[pl.ds(off_of(c3), ch)], idx[b3], isem[b3]
                    ).wait()
                    adjust(b3)
                    if b == 0:
                        @pl.when(jq >= 1)
                        def _():
                            pltpu.make_async_copy(
                                rows[b3], out_hbm.at[pl.ds(0, ch)], osem[b3]
                            ).wait()
                    else:
                        pltpu.make_async_copy(
                            rows[b3], out_hbm.at[pl.ds(0, ch)], osem[b3]
                        ).wait()
                    pltpu.async_copy(wt_hbm.at[idx[b3]], rows[b3], gsem[b3])

            return carry

        lax.fori_loop(0, n_iters // nbuf, body, 0)

        for b in range(nbuf):
            pltpu.make_async_copy(
                rows[b], out_hbm.at[pl.ds(0, ch)], osem[b]).wait()

    return _gather(wt, x)
